# baseline (speedup 1.0000x reference)
import functools

import jax
import jax.numpy as jnp
from jax import lax
from jax.experimental import pallas as pl
from jax.experimental.pallas import tpu as pltpu
from jax.experimental.pallas import tpu_sc as plsc

NUM_EMBEDDINGS = 1000000
EMBED_DIM = 32
NUM_FEATURES = 2
BATCH = 4096
MAX_LEN = 20

LANES = 16
NUM_WORKERS = 32
TOTAL_ENT = NUM_FEATURES * BATCH * MAX_LEN
PAD_ENT = 256
CW = 512
NFULL = NUM_EMBEDDINGS // CW
TAIL_BASE = NFULL * CW
TAIL_W = NUM_EMBEDDINGS - TAIL_BASE
KMAX = 62
NBATCH = 128
FKEY = 1 << 20
NSLOT = NUM_FEATURES * BATCH
BROW = 256


def _sel(vec16, lane):
    io = jax.lax.broadcasted_iota(jnp.int32, (LANES,), 0)
    return jnp.sum(jnp.where(io == lane, vec16, 0), axis=0)


def _zero16():
    return jnp.zeros((LANES,), jnp.float32)


def _run_batches(skey_hbm, smeta_hbm, chunk_v, skey_v, smeta_v, wv_v,
                 contrib_v, slot_v, acc_sh, f, cbase, cw, lo, hi):
    s0 = lo - lax.rem(lo, NBATCH)
    nb = lax.div(hi - s0 + (NBATCH - 1), NBATCH)
    iota16 = jax.lax.broadcasted_iota(jnp.int32, (LANES,), 0)

    def batch_body(b, carry):
        estart = pl.multiple_of(s0 + b * NBATCH, NBATCH)
        pltpu.sync_copy(skey_hbm.at[pl.ds(estart, NBATCH)], skey_v)
        pltpu.sync_copy(smeta_hbm.at[pl.ds(estart, NBATCH)], smeta_v)

        def group_body(g, gc):
            gbase = pl.multiple_of(g * LANES, LANES)
            k16 = skey_v[pl.ds(gbase, LANES)]
            m16 = smeta_v[pl.ds(gbase, LANES)]
            pos16 = estart + g * LANES + iota16
            inr = jnp.logical_and(pos16 >= lo, pos16 < hi)
            off16 = jnp.clip(k16 - (f * FKEY + cbase), 0, cw - 1)
            wslot = f * MAX_LEN + jnp.bitwise_and(m16, 31)
            wraw = plsc.load_gather(
                wv_v, [jnp.zeros((LANES,), jnp.int32), wslot])
            w16 = jnp.where(inr, wraw, 0.0)
            slot_v[pl.ds(gbase, LANES)] = (
                f * BATCH + jax.lax.shift_right_logical(m16, 5))
            rowidx = iota16 + g * LANES
            for d in range(EMBED_DIM):
                v16 = plsc.load_gather(
                    chunk_v, [jnp.full((LANES,), d, jnp.int32), off16])
                plsc.store_scatter(
                    contrib_v,
                    [rowidx, jnp.full((LANES,), d, jnp.int32)],
                    v16 * w16)
            return gc

        lax.fori_loop(0, NBATCH // LANES, group_body, 0)
        pltpu.sync_copy(contrib_v, acc_sh.at[slot_v], add=True)
        return carry

    lax.fori_loop(0, nb, batch_body, 0)


def _sc_body(skey_hbm, smeta_hbm, bndw_hbm, wexp_hbm, tT0, tT1,
             tl0_hbm, tl1_hbm, out_hbm,
             chunk_v, skey_v, smeta_v, wv_v, contrib_v, slot_v, bnd_v,
             acc_sh):
    cid = lax.axis_index("c")
    sid = lax.axis_index("s")
    wid = sid * 2 + cid

    pltpu.sync_copy(wexp_hbm, wv_v)
    pltpu.sync_copy(bndw_hbm.at[pl.ds(wid * BROW, BROW)], bnd_v)

    def crow(i, c):
        for q in range(128 // LANES):
            contrib_v[i, pl.ds(q * LANES, LANES)] = _zero16()
        return c
    lax.fori_loop(0, NBATCH, crow, 0)
    for r in range(NSLOT // 16 // NBATCH):
        pltpu.sync_copy(
            contrib_v, acc_sh.at[pl.ds(sid * (NSLOT // 16) + r * NBATCH,
                                       NBATCH)])
    plsc.subcore_barrier()

    for f in range(NUM_FEATURES):
        tab = tT0 if f == 0 else tT1

        def kstep(k, carry):
            j = k * 2
            al = j - lax.rem(j, LANES)
            grp = bnd_v[pl.ds(f * 128 + al, LANES)]
            lane = j - al
            lo = _sel(grp, lane)
            hi = _sel(grp, lane + 1)
            cbase = pl.multiple_of((wid + k * NUM_WORKERS) * CW, CW)
            pltpu.sync_copy(tab.at[:, pl.ds(cbase, CW)], chunk_v)
            _run_batches(skey_hbm, smeta_hbm, chunk_v, skey_v, smeta_v,
                         wv_v, contrib_v, slot_v, acc_sh,
                         f, cbase, CW, lo, hi)
            return carry

        nk = lax.div(NFULL - 1 - wid, NUM_WORKERS) + 1
        lax.fori_loop(0, nk, kstep, 0)

        @pl.when(wid == 9)
        def _():
            grp = bnd_v[pl.ds(f * 128 + 112, LANES)]
            tl = tl0_hbm if f == 0 else tl1_hbm
            pltpu.sync_copy(tl, chunk_v.at[:, pl.ds(0, 128)])
            _run_batches(skey_hbm, smeta_hbm, chunk_v, skey_v, smeta_v,
                         wv_v, contrib_v, slot_v, acc_sh,
                         f, TAIL_BASE, 128, grp[12], grp[13])

    plsc.subcore_barrier()
    pltpu.sync_copy(
        acc_sh.at[pl.ds(sid * 512, 512)],
        out_hbm.at[pl.ds(cid * NSLOT + sid * 512, 512)])


def _tc_sum_body(p_ref, o_ref):
    for f in range(NUM_FEATURES):
        s = p_ref[0, f] + p_ref[1, f]
        o_ref[:, f * EMBED_DIM:(f + 1) * EMBED_DIM] = s[:, :EMBED_DIM]


@jax.jit
def _fpebc(skey, smeta, bndw, wexp, tT0, tT1, tl0, tl1):
    mesh = plsc.VectorSubcoreMesh(core_axis_name="c", subcore_axis_name="s")
    kern = functools.partial(
        pl.kernel,
        out_type=jax.ShapeDtypeStruct((2 * NSLOT, 128), jnp.float32),
        mesh=mesh,
        compiler_params=pltpu.CompilerParams(use_tc_tiling_on_sc=True,
                                             needs_layout_passes=False),
        scratch_types=[
            pltpu.VMEM((EMBED_DIM, CW), jnp.float32),
            pltpu.VMEM((NBATCH,), jnp.int32),
            pltpu.VMEM((NBATCH,), jnp.int32),
            pltpu.VMEM((1, NUM_FEATURES * MAX_LEN), jnp.float32),
            pltpu.VMEM((NBATCH, 128), jnp.float32),
            pltpu.VMEM((NBATCH,), jnp.int32),
            pltpu.VMEM((BROW,), jnp.int32),
            pltpu.VMEM_SHARED((NSLOT, 128), jnp.float32),
        ],
    )(_sc_body)
    partials = kern(skey, smeta, bndw, wexp, tT0, tT1, tl0, tl1)
    p = partials.reshape(2, NUM_FEATURES, BATCH, 128)
    out = pl.pallas_call(
        _tc_sum_body,
        out_shape=jax.ShapeDtypeStruct((BATCH, NUM_FEATURES * EMBED_DIM),
                                       jnp.float32),
    )(p)
    return out


def kernel(indices, table0, table1, pos_w):
    f_ar = jnp.arange(NUM_FEATURES, dtype=jnp.int32)[:, None, None]
    b_ar = jnp.arange(BATCH, dtype=jnp.int32)[None, :, None]
    l_ar = jnp.arange(MAX_LEN, dtype=jnp.int32)[None, None, :]
    keys = (f_ar * FKEY + indices).reshape(-1)
    meta = jnp.broadcast_to(b_ar * 32 + l_ar,
                            (NUM_FEATURES, BATCH, MAX_LEN)).reshape(-1)
    order = jnp.argsort(keys)
    skey = jnp.concatenate(
        [keys[order], jnp.full((PAD_ENT,), 2 * FKEY, jnp.int32)])
    smeta = jnp.concatenate([meta[order], jnp.zeros((PAD_ENT,), jnp.int32)])

    npc = NFULL + 2
    edges = (f_ar[:, 0, 0][:, None] * FKEY
             + jnp.concatenate(
                 [jnp.arange(NFULL, dtype=jnp.int32) * CW,
                  jnp.array([TAIL_BASE, NUM_EMBEDDINGS], jnp.int32)])
             ).reshape(-1)
    bnd = jnp.searchsorted(skey[:TOTAL_ENT], edges,
                           side="left").astype(jnp.int32)
    bndw = jnp.zeros((NUM_WORKERS, BROW), jnp.int32)
    w_ar = jnp.arange(NUM_WORKERS, dtype=jnp.int32)
    for f in range(NUM_FEATURES):
        for k in range(KMAX):
            cc = jnp.minimum(w_ar + k * NUM_WORKERS, NFULL - 1)
            bndw = bndw.at[:, f * 128 + 2 * k].set(bnd[f * npc + cc])
            bndw = bndw.at[:, f * 128 + 2 * k + 1].set(bnd[f * npc + cc + 1])
        bndw = bndw.at[:, f * 128 + 124].set(bnd[f * npc + NFULL])
        bndw = bndw.at[:, f * 128 + 125].set(bnd[f * npc + NFULL + 1])
    wexp = pos_w.reshape(1, NUM_FEATURES * MAX_LEN).astype(jnp.float32)
    tl0 = jnp.zeros((EMBED_DIM, 128), jnp.float32).at[:, :TAIL_W].set(
        table0[TAIL_BASE:].T)
    tl1 = jnp.zeros((EMBED_DIM, 128), jnp.float32).at[:, :TAIL_W].set(
        table1[TAIL_BASE:].T)
    return _fpebc(skey, smeta, bndw.reshape(-1), wexp, table0.T, table1.T,
                  tl0, tl1)


# speedup vs baseline: 3.5223x; 3.5223x over previous
import functools

import jax
import jax.numpy as jnp
from jax import lax
from jax.experimental import pallas as pl
from jax.experimental.pallas import tpu as pltpu
from jax.experimental.pallas import tpu_sc as plsc

NUM_EMBEDDINGS = 1000000
EMBED_DIM = 32
NUM_FEATURES = 2
BATCH = 4096
MAX_LEN = 20

LANES = 16
NUM_WORKERS = 32
BAGS_PER_WORKER = BATCH // NUM_WORKERS
IDX_PER_WORKER = BAGS_PER_WORKER * MAX_LEN
GATHER_BATCH = 128
GATHERS_PER_FEATURE = IDX_PER_WORKER // GATHER_BATCH


def _sc_body(idx_hbm, t0_hbm, t1_hbm, wv_hbm, out_hbm,
             idx_v, rows_v, acc_v, wv_v, sem):
    cid = lax.axis_index("c")
    sid = lax.axis_index("s")
    wid = sid * 2 + cid
    base_bag = wid * BAGS_PER_WORKER

    pltpu.sync_copy(wv_hbm, wv_v)

    for f in range(NUM_FEATURES):
        table = t0_hbm if f == 0 else t1_hbm

        pltpu.sync_copy(
            idx_hbm.at[pl.ds(f * BATCH * MAX_LEN + wid * IDX_PER_WORKER,
                             IDX_PER_WORKER)],
            idx_v)

        copies = []
        for j in range(GATHERS_PER_FEATURE):
            copies.append(
                pltpu.async_copy(
                    table.at[idx_v.at[pl.ds(j * GATHER_BATCH, GATHER_BATCH)]],
                    rows_v.at[pl.ds(j * GATHER_BATCH, GATHER_BATCH)],
                    sem))
        for c in copies:
            c.wait()

        w = tuple(wv_v[pl.ds((f * MAX_LEN + l) * LANES, LANES)]
                  for l in range(MAX_LEN))

        def row2(r):
            packed = rows_v[r, pl.ds(0, 2 * LANES)]
            return plsc.unpack(packed, format=plsc.PackFormat.INTERLEAVED)

        def bag_body(b, w):
            r0 = b * MAX_LEN
            x0, x1 = row2(r0)
            a0e = w[0] * x0
            a1e = w[0] * x1
            y0, y1 = row2(r0 + 1)
            a0o = w[1] * y0
            a1o = w[1] * y1
            for l in range(2, MAX_LEN, 2):
                x0, x1 = row2(r0 + l)
                a0e = a0e + w[l] * x0
                a1e = a1e + w[l] * x1
                y0, y1 = row2(r0 + l + 1)
                a0o = a0o + w[l + 1] * y0
                a1o = a1o + w[l + 1] * y1
            o = pl.multiple_of(b * EMBED_DIM, EMBED_DIM)
            acc_v[pl.ds(o, LANES)] = a0e + a0o
            acc_v[pl.ds(o + LANES, LANES)] = a1e + a1o
            return w

        lax.fori_loop(0, BAGS_PER_WORKER, bag_body, w, unroll=False)

        pltpu.sync_copy(
            acc_v,
            out_hbm.at[pl.ds(f * BATCH * EMBED_DIM + base_bag * EMBED_DIM,
                             BAGS_PER_WORKER * EMBED_DIM)])


@jax.jit
def _fpebc(idx_flat, table0, table1, wv):
    mesh = plsc.VectorSubcoreMesh(core_axis_name="c", subcore_axis_name="s")
    kern = functools.partial(
        pl.kernel,
        out_type=jax.ShapeDtypeStruct((NUM_FEATURES * BATCH * EMBED_DIM,),
                                      jnp.float32),
        mesh=mesh,
        compiler_params=pltpu.CompilerParams(use_tc_tiling_on_sc=False,
                                             needs_layout_passes=False),
        scratch_types=[
            pltpu.VMEM((IDX_PER_WORKER,), jnp.int32),
            pltpu.VMEM((IDX_PER_WORKER, EMBED_DIM), jnp.bfloat16),
            pltpu.VMEM((BAGS_PER_WORKER * EMBED_DIM,), jnp.float32),
            pltpu.VMEM((NUM_FEATURES * MAX_LEN * LANES,), jnp.float32),
            pltpu.SemaphoreType.DMA,
        ],
    )(_sc_body)
    out_flat = kern(idx_flat, table0, table1, wv)
    stored = (out_flat.reshape(NUM_FEATURES, BATCH, EMBED_DIM)
              .transpose(1, 0, 2)
              .reshape(BATCH, NUM_FEATURES * EMBED_DIM))
    pos = [(d // 2) if d % 2 == 0 else (16 + (d - 1) // 2)
           for d in range(EMBED_DIM)]
    cols = jnp.array([f * EMBED_DIM + p for f in range(NUM_FEATURES)
                      for p in pos], jnp.int32)
    return stored[:, cols]


def kernel(indices, table0, table1, pos_w):
    idx_flat = indices.reshape(-1)
    table0 = table0.astype(jnp.bfloat16)
    table1 = table1.astype(jnp.bfloat16)
    wv = jnp.broadcast_to(pos_w[:, :, None],
                          (NUM_FEATURES, MAX_LEN, LANES)).reshape(-1)
    return _fpebc(idx_flat, table0, table1, wv)


# speedup vs baseline: 4.1075x; 1.1662x over previous
import functools

import jax
import jax.numpy as jnp
from jax import lax
from jax.experimental import pallas as pl
from jax.experimental.pallas import tpu as pltpu
from jax.experimental.pallas import tpu_sc as plsc

NUM_EMBEDDINGS = 1000000
EMBED_DIM = 32
NUM_FEATURES = 2
BATCH = 4096
MAX_LEN = 20

LANES = 16
NUM_WORKERS = 32
BAGS_PER_WORKER = BATCH // NUM_WORKERS
IDX_PER_WORKER = BAGS_PER_WORKER * MAX_LEN
GATHER_BATCH = 128
GATHERS_PER_FEATURE = IDX_PER_WORKER // GATHER_BATCH


def _sc_body(idx_hbm, t0_hbm, t1_hbm, wv_hbm, out_hbm,
             idx_v, rows_v, acc_v, wv_v, sem):
    cid = lax.axis_index("c")
    sid = lax.axis_index("s")
    wid = sid * 2 + cid
    base_bag = wid * BAGS_PER_WORKER

    pltpu.sync_copy(wv_hbm, wv_v)

    for f in range(NUM_FEATURES):
        table = t0_hbm if f == 0 else t1_hbm

        pltpu.sync_copy(
            idx_hbm.at[pl.ds(f * BATCH * MAX_LEN + wid * IDX_PER_WORKER,
                             IDX_PER_WORKER)],
            idx_v)

        copies = []
        for j in range(GATHERS_PER_FEATURE):
            copies.append(
                pltpu.async_copy(
                    table.at[idx_v.at[pl.ds(j * GATHER_BATCH, GATHER_BATCH)]],
                    rows_v.at[pl.ds(j * GATHER_BATCH, GATHER_BATCH)],
                    sem))
        for c in copies:
            c.wait()

        w = tuple(wv_v[pl.ds((f * MAX_LEN + l) * LANES, LANES)]
                  for l in range(MAX_LEN))

        def bag_body(b, w):
            r0 = b * MAX_LEN
            a0e = w[0] * rows_v[r0, pl.ds(0, LANES)]
            a1e = w[0] * rows_v[r0, pl.ds(LANES, LANES)]
            a0o = w[1] * rows_v[r0 + 1, pl.ds(0, LANES)]
            a1o = w[1] * rows_v[r0 + 1, pl.ds(LANES, LANES)]
            for l in range(2, MAX_LEN, 2):
                a0e = a0e + w[l] * rows_v[r0 + l, pl.ds(0, LANES)]
                a1e = a1e + w[l] * rows_v[r0 + l, pl.ds(LANES, LANES)]
                a0o = a0o + w[l + 1] * rows_v[r0 + l + 1, pl.ds(0, LANES)]
                a1o = a1o + w[l + 1] * rows_v[r0 + l + 1, pl.ds(LANES, LANES)]
            o = pl.multiple_of(b * EMBED_DIM, EMBED_DIM)
            acc_v[pl.ds(o, LANES)] = a0e + a0o
            acc_v[pl.ds(o + LANES, LANES)] = a1e + a1o
            return w

        lax.fori_loop(0, BAGS_PER_WORKER, bag_body, w, unroll=False)

        pltpu.sync_copy(
            acc_v,
            out_hbm.at[pl.ds(f * BATCH * EMBED_DIM + base_bag * EMBED_DIM,
                             BAGS_PER_WORKER * EMBED_DIM)])


@jax.jit
def _fpebc(idx_flat, table0, table1, wv):
    mesh = plsc.VectorSubcoreMesh(core_axis_name="c", subcore_axis_name="s")
    kern = functools.partial(
        pl.kernel,
        out_type=jax.ShapeDtypeStruct((NUM_FEATURES * BATCH * EMBED_DIM,),
                                      jnp.float32),
        mesh=mesh,
        compiler_params=pltpu.CompilerParams(use_tc_tiling_on_sc=False),
        scratch_types=[
            pltpu.VMEM((IDX_PER_WORKER,), jnp.int32),
            pltpu.VMEM((IDX_PER_WORKER, EMBED_DIM), jnp.float32),
            pltpu.VMEM((BAGS_PER_WORKER * EMBED_DIM,), jnp.float32),
            pltpu.VMEM((NUM_FEATURES * MAX_LEN * LANES,), jnp.float32),
            pltpu.SemaphoreType.DMA,
        ],
    )(_sc_body)
    out_flat = kern(idx_flat, table0, table1, wv)
    return (out_flat.reshape(NUM_FEATURES, BATCH, EMBED_DIM)
            .transpose(1, 0, 2)
            .reshape(BATCH, NUM_FEATURES * EMBED_DIM))


def kernel(indices, table0, table1, pos_w):
    idx_flat = indices.reshape(-1)
    wv = jnp.broadcast_to(pos_w[:, :, None],
                          (NUM_FEATURES, MAX_LEN, LANES)).reshape(-1)
    return _fpebc(idx_flat, table0, table1, wv)
